# baseline (speedup 1.0000x reference)
import functools

import jax
import jax.numpy as jnp
from jax import lax
from jax.experimental import pallas as pl
from jax.experimental.pallas import tpu as pltpu
from jax.experimental.pallas import tpu_sc as plsc

N = 10000
E = 320000
D_IN = 3
D_X = 2
L = 128
STEPS = 6

N_PAD = 10240
E_PAD = 327680
NC = 2
NS = 16
NW = NC * NS
EPW = E_PAD // NW
K = 128
C = EPW // K
TOT = 2 * C
S = 5
RPS = N_PAD // NS
ZCH = RPS // K

_F32 = jnp.float32


def _mesh():
    return plsc.VectorSubcoreMesh(core_axis_name="c", subcore_axis_name="s",
                                  num_cores=NC, num_subcores=NS)


def _sc_gather2(t2, idx2):
    D = t2.shape[-1]

    @functools.partial(
        pl.kernel,
        out_type=jax.ShapeDtypeStruct((2 * E_PAD, D), _F32),
        mesh=_mesh(),
        scratch_types=[pltpu.VMEM((TOT, K), jnp.int32)]
        + [pltpu.VMEM((K, D), _F32)] * S
        + [pltpu.SemaphoreType.DMA] * (2 * S),
    )
    def k(t2_h, i2_h, out_h, iva, *rest):
        slots = rest[:S]
        gsem = rest[S:2 * S]
        wsem = rest[2 * S:]
        wid = lax.axis_index("c") * NS + lax.axis_index("s")
        pltpu.sync_copy(i2_h.at[wid], iva)

        def off_of(j):
            return (wid * EPW + j * K
                    + jnp.where(j >= C, E_PAD - EPW, 0).astype(jnp.int32))

        def body(g, carry):
            gds = [pltpu.async_copy(t2_h.at[iva.at[g * S]], slots[0],
                                    gsem[0])]
            wds = []
            for b in range(S):
                if b + 1 < S:
                    gds.append(pltpu.async_copy(
                        t2_h.at[iva.at[g * S + b + 1]], slots[b + 1],
                        gsem[b + 1]))
                gds[b].wait()
                wds.append(pltpu.async_copy(
                    slots[b], out_h.at[pl.ds(off_of(g * S + b), K)],
                    wsem[b]))
            for wd in wds:
                wd.wait()
            return carry

        lax.fori_loop(0, TOT // S, body, 0)

    return k(t2, idx2)


def _sc_segment_sum(e_all, rcv3):
    SS = 2

    @functools.partial(
        pl.kernel,
        out_type=jax.ShapeDtypeStruct((NC, N_PAD, L), _F32),
        mesh=_mesh(),
        scratch_types=[pltpu.VMEM((C, K), jnp.int32)]
        + [pltpu.VMEM((K, L), _F32)] * SS
        + [pltpu.SemaphoreType.DMA] * (2 * SS)
        + [pltpu.VMEM_SHARED((N_PAD, L), _F32)],
    )
    def k(e_h, r_h, out_h, idxv, *rest):
        slots = rest[:SS]
        rsem = rest[SS:2 * SS]
        ssem = rest[2 * SS:3 * SS]
        acc = rest[3 * SS]
        cid = lax.axis_index("c")
        sid = lax.axis_index("s")
        wid = cid * NS + sid
        pltpu.sync_copy(r_h.at[wid], idxv)

        def zrow(i, carry):
            def zcol(j, c2):
                slots[0][i, pl.ds(j * 16, 16)] = jnp.zeros((16,), _F32)
                return c2
            return lax.fori_loop(0, L // 16, zcol, carry)

        lax.fori_loop(0, K, zrow, 0)
        row0 = sid * RPS

        def zc(i, carry):
            pltpu.sync_copy(slots[0], acc.at[pl.ds(row0 + i * K, K)])
            return carry

        lax.fori_loop(0, ZCH, zc, 0)
        plsc.subcore_barrier()

        ebase = wid * EPW

        def body(g, carry):
            rds = [pltpu.async_copy(
                e_h.at[pl.ds(ebase + (g * SS + b) * K, K)], slots[b],
                rsem[b]) for b in range(SS)]
            sds = []
            for b in range(SS):
                rds[b].wait()
                sds.append(pltpu.async_copy(
                    slots[b], acc.at[idxv.at[g * SS + b]], ssem[b],
                    add=True))
            for sd in sds:
                sd.wait()
            return carry

        lax.fori_loop(0, C // SS, body, 0)
        plsc.subcore_barrier()

        def outc(i, carry):
            pltpu.sync_copy(acc.at[pl.ds(row0 + i * K, K)], slots[0])
            pltpu.sync_copy(slots[0], out_h.at[cid, pl.ds(row0 + i * K, K)])
            return carry

        lax.fori_loop(0, ZCH, outc, 0)

    return k(e_all, rcv3)


def _dot(a, b):
    return jnp.dot(a, b, preferred_element_type=_F32)


_NBLK = 2048
_EBLK = 2048


def _node_specs(d):
    return pl.BlockSpec((_NBLK, d), lambda i: (i, 0))


def _edge_specs(d):
    return pl.BlockSpec((_EBLK, d), lambda i: (i, 0))


def _w_spec(shape):
    return pl.BlockSpec(shape, lambda i: tuple(0 for _ in shape))


_P2_SPEC = pl.BlockSpec((2, _NBLK, L), lambda i: (0, i, 0))
_GR_SPEC = pl.BlockSpec((_EBLK, L), lambda i: (i + E_PAD // _EBLK, 0))


def _tc_node_encode(u8, Wn1p, bn1, Wn2, bn2, Wms, Wmr, bm):
    def body(u_r, w1_r, b1_r, w2_r, b2_r, wms_r, wmr_r, bm_r, h_r, p2_r):
        t = jnp.tanh(_dot(u_r[...], w1_r[...]) + b1_r[...])
        h = _dot(t, w2_r[...]) + b2_r[...]
        h_r[...] = h
        p2_r[0] = _dot(h, wms_r[...]) + bm_r[...]
        p2_r[1] = _dot(h, wmr_r[...])

    return pl.pallas_call(
        body,
        grid=(N_PAD // _NBLK,),
        in_specs=[_node_specs(8), _w_spec((8, L)), _w_spec((1, L)),
                  _w_spec((L, L)), _w_spec((1, L)), _w_spec((L, L)),
                  _w_spec((L, L)), _w_spec((1, L))],
        out_specs=[_node_specs(L), _P2_SPEC],
        out_shape=[jax.ShapeDtypeStruct((N_PAD, L), _F32),
                   jax.ShapeDtypeStruct((2, N_PAD, L), _F32)],
    )(u8, Wn1p, bn1, Wn2, bn2, Wms, Wmr, bm)


def _tc_edge_encode(x2g, We1p, be1, We2, be2):
    def body(xs_r, xr_r, w1_r, b1_r, w2_r, b2_r, e_r):
        z = xs_r[...] - xr_r[...]
        z = jnp.where(z >= 1.0, z - 2.0, z)
        z = jnp.where(z < -1.0, z + 2.0, z)
        d = jnp.sqrt(jnp.sum(z * z, axis=1, keepdims=True))
        lane = lax.broadcasted_iota(jnp.int32, z.shape, 1)
        za = jnp.where(lane == 2, d, z)
        f = _dot(za, w1_r[...]) + b1_r[...]
        e_r[...] = _dot(jnp.tanh(f), w2_r[...]) + b2_r[...]

    return pl.pallas_call(
        body,
        grid=(E_PAD // _EBLK,),
        in_specs=[_edge_specs(L), _GR_SPEC, _w_spec((L, L)),
                  _w_spec((1, L)), _w_spec((L, L)), _w_spec((1, L))],
        out_specs=_edge_specs(L),
        out_shape=jax.ShapeDtypeStruct((E_PAD, L), _F32),
    )(x2g, x2g, We1p, be1, We2, be2)


def _tc_edge_update(e, g2, Wme):
    def body(e_r, gs_r, gr_r, w_r, o_r):
        o_r[...] = e_r[...] + jnp.tanh(
            _dot(e_r[...], w_r[...]) + gs_r[...] + gr_r[...])

    return pl.pallas_call(
        body,
        grid=(E_PAD // _EBLK,),
        in_specs=[_edge_specs(L), _edge_specs(L), _GR_SPEC, _w_spec((L, L))],
        out_specs=_edge_specs(L),
        out_shape=jax.ShapeDtypeStruct((E_PAD, L), _F32),
    )(e, g2, g2, Wme)


def _tc_node_update(h, agg, Wuh, Wua, bu, Wms, Wmr, bm):
    def body(h_r, a0_r, a1_r, wh_r, wa_r, bu_r, wms_r, wmr_r, bm_r,
             hn_r, p2_r):
        hn = h_r[...] + jnp.tanh(
            _dot(h_r[...], wh_r[...]) + _dot(a0_r[0] + a1_r[0], wa_r[...])
            + bu_r[...])
        hn_r[...] = hn
        p2_r[0] = _dot(hn, wms_r[...]) + bm_r[...]
        p2_r[1] = _dot(hn, wmr_r[...])

    aspec = pl.BlockSpec((1, _NBLK, L), lambda i: (0, i, 0))
    return pl.pallas_call(
        body,
        grid=(N_PAD // _NBLK,),
        in_specs=[_node_specs(L), aspec, aspec, _w_spec((L, L)),
                  _w_spec((L, L)), _w_spec((1, L)), _w_spec((L, L)),
                  _w_spec((L, L)), _w_spec((1, L))],
        out_specs=[_node_specs(L), _P2_SPEC],
        out_shape=[jax.ShapeDtypeStruct((N_PAD, L), _F32),
                   jax.ShapeDtypeStruct((2, N_PAD, L), _F32)],
    )(h, agg[:1], agg[1:], Wuh, Wua, bu, Wms, Wmr, bm)


def _tc_node_final(h, agg, Wuh, Wua, bu, Wd1, bd1, Wd2p, bd2p):
    def body(h_r, a0_r, a1_r, wh_r, wa_r, bu_r, w1_r, b1_r, w2_r, b2_r, o_r):
        hn = h_r[...] + jnp.tanh(
            _dot(h_r[...], wh_r[...]) + _dot(a0_r[0] + a1_r[0], wa_r[...])
            + bu_r[...])
        t = jnp.tanh(_dot(hn, w1_r[...]) + b1_r[...])
        o_r[...] = _dot(t, w2_r[...]) + b2_r[...]

    aspec = pl.BlockSpec((1, _NBLK, L), lambda i: (0, i, 0))
    return pl.pallas_call(
        body,
        grid=(N_PAD // _NBLK,),
        in_specs=[_node_specs(L), aspec, aspec, _w_spec((L, L)),
                  _w_spec((L, L)), _w_spec((1, L)), _w_spec((L, L)),
                  _w_spec((1, L)), _w_spec((L, 8)), _w_spec((1, 8))],
        out_specs=_node_specs(8),
        out_shape=jax.ShapeDtypeStruct((N_PAD, 8), _F32),
    )(h, agg[:1], agg[1:], Wuh, Wua, bu, Wd1, bd1, Wd2p, bd2p)


def kernel(u_inp, x_inp, edge_index, params):
    p = params
    row = lambda v: v.reshape(1, -1)

    u8 = jnp.pad(u_inp[0, 0], ((0, N_PAD - N), (0, 8 - D_IN)))
    x128 = jnp.pad(x_inp, ((0, N_PAD - N), (0, L - D_X)))
    x2 = jnp.concatenate([x128, x128], axis=0)
    pad_e = E_PAD - E
    snd3 = jnp.pad(edge_index[0], (0, pad_e)).reshape(NW, C, K)
    rcv3 = jnp.pad(edge_index[1], (0, pad_e),
                   constant_values=N).reshape(NW, C, K)
    idx2 = jnp.concatenate([snd3, rcv3 + N_PAD], axis=1)

    Wn1p = jnp.pad(p["Wn1"], ((0, 8 - D_IN), (0, 0)))
    We1p = jnp.pad(p["We1"], ((0, L - D_X - 1), (0, 0)))
    Wm_e = p["Wm"][:, :L]
    Wm_s = p["Wm"][:, L:2 * L]
    Wm_r = p["Wm"][:, 2 * L:]
    Wu_h = p["Wu"][:, :L]
    Wu_a = p["Wu"][:, L:]
    Wd2p = jnp.pad(p["Wd2"], ((0, 0), (0, 8 - D_IN)))
    bd2p = row(jnp.pad(p["bd2"], (0, 8 - D_IN)))

    h, p2 = _tc_node_encode(u8, Wn1p, row(p["bn1"]), p["Wn2"],
                            row(p["bn2"]), Wm_s[0], Wm_r[0],
                            row(p["bm"][0]))
    x2g = _sc_gather2(x2, idx2)
    e = _tc_edge_encode(x2g, We1p, row(p["be1"]), p["We2"], row(p["be2"]))

    for s in range(STEPS):
        g2 = _sc_gather2(p2.reshape(2 * N_PAD, L), idx2)
        e = _tc_edge_update(e, g2, Wm_e[s])
        agg = _sc_segment_sum(e, rcv3)
        if s + 1 < STEPS:
            h, p2 = _tc_node_update(h, agg, Wu_h[s], Wu_a[s],
                                    row(p["bu"][s]), Wm_s[s + 1],
                                    Wm_r[s + 1], row(p["bm"][s + 1]))
        else:
            out = _tc_node_final(h, agg, Wu_h[s], Wu_a[s], row(p["bu"][s]),
                                 p["Wd1"], row(p["bd1"]), Wd2p, bd2p)

    return out[:N, :D_IN]


# speedup vs baseline: 1.2889x; 1.2889x over previous
import functools

import jax
import jax.numpy as jnp
from jax import lax
from jax.experimental import pallas as pl
from jax.experimental.pallas import tpu as pltpu
from jax.experimental.pallas import tpu_sc as plsc

N = 10000
E = 320000
D_IN = 3
D_X = 2
L = 128
STEPS = 6

N_PAD = 10240
E_PAD = 327680
NC = 2
NS = 16
NW = NC * NS
EPW = E_PAD // NW
K = 128
C = EPW // K
TOT = 2 * C
S = 5
RPS = N_PAD // NS
ZCH = RPS // K

_F32 = jnp.float32


def _mesh():
    return plsc.VectorSubcoreMesh(core_axis_name="c", subcore_axis_name="s",
                                  num_cores=NC, num_subcores=NS)


def _sc_gather2(t2, idx2):
    D = t2.shape[-1]

    @functools.partial(
        pl.kernel,
        out_type=jax.ShapeDtypeStruct((2 * E_PAD, D), _F32),
        mesh=_mesh(),
        scratch_types=[
            pltpu.VMEM((TOT, K), jnp.int32),
            pltpu.VMEM((K, D), _F32),
            pltpu.VMEM((K, D), _F32),
            pltpu.SemaphoreType.DMA,
            pltpu.SemaphoreType.DMA,
        ],
    )
    def k(t2_h, i2_h, out_h, iva, ba, bb, sa, sb):
        wid = lax.axis_index("c") * NS + lax.axis_index("s")
        pltpu.sync_copy(i2_h.at[wid], iva)
        base_s = wid * EPW
        base_r = E_PAD + wid * EPW

        def body(j, carry):
            cpa = pltpu.async_copy(t2_h.at[iva.at[j]], ba, sa)
            cpb = pltpu.async_copy(t2_h.at[iva.at[j + C]], bb, sb)
            cpa.wait()
            pltpu.sync_copy(ba, out_h.at[pl.ds(base_s + j * K, K)])
            cpb.wait()
            pltpu.sync_copy(bb, out_h.at[pl.ds(base_r + j * K, K)])
            return carry

        lax.fori_loop(0, C, body, 0)

    return k(t2, idx2)


def _sc_segment_sum(e_all, rcv3):
    SS = 2

    @functools.partial(
        pl.kernel,
        out_type=jax.ShapeDtypeStruct((NC, N_PAD, L), _F32),
        mesh=_mesh(),
        scratch_types=[pltpu.VMEM((C, K), jnp.int32)]
        + [pltpu.VMEM((K, L), _F32)] * SS
        + [pltpu.SemaphoreType.DMA] * (2 * SS)
        + [pltpu.VMEM_SHARED((N_PAD, L), _F32)],
    )
    def k(e_h, r_h, out_h, idxv, *rest):
        slots = rest[:SS]
        rsem = rest[SS:2 * SS]
        ssem = rest[2 * SS:3 * SS]
        acc = rest[3 * SS]
        cid = lax.axis_index("c")
        sid = lax.axis_index("s")
        wid = cid * NS + sid
        pltpu.sync_copy(r_h.at[wid], idxv)

        def zrow(i, carry):
            def zcol(j, c2):
                slots[0][i, pl.ds(j * 16, 16)] = jnp.zeros((16,), _F32)
                return c2
            return lax.fori_loop(0, L // 16, zcol, carry)

        lax.fori_loop(0, K, zrow, 0)
        row0 = sid * RPS

        def zc(i, carry):
            pltpu.sync_copy(slots[0], acc.at[pl.ds(row0 + i * K, K)])
            return carry

        lax.fori_loop(0, ZCH, zc, 0)
        plsc.subcore_barrier()

        ebase = wid * EPW

        def body(g, carry):
            rds = [pltpu.async_copy(
                e_h.at[pl.ds(ebase + (g * SS + b) * K, K)], slots[b],
                rsem[b]) for b in range(SS)]
            sds = []
            for b in range(SS):
                rds[b].wait()
                sds.append(pltpu.async_copy(
                    slots[b], acc.at[idxv.at[g * SS + b]], ssem[b],
                    add=True))
            for sd in sds:
                sd.wait()
            return carry

        lax.fori_loop(0, C // SS, body, 0)
        plsc.subcore_barrier()

        def outc(i, carry):
            pltpu.sync_copy(acc.at[pl.ds(row0 + i * K, K)], slots[0])
            pltpu.sync_copy(slots[0], out_h.at[cid, pl.ds(row0 + i * K, K)])
            return carry

        lax.fori_loop(0, ZCH, outc, 0)

    return k(e_all, rcv3)


def _dot(a, b):
    return jnp.dot(a, b, preferred_element_type=_F32)


_NBLK = 2048
_EBLK = 2048


def _node_specs(d):
    return pl.BlockSpec((_NBLK, d), lambda i: (i, 0))


def _edge_specs(d):
    return pl.BlockSpec((_EBLK, d), lambda i: (i, 0))


def _w_spec(shape):
    return pl.BlockSpec(shape, lambda i: tuple(0 for _ in shape))


_P2_SPEC = pl.BlockSpec((2, _NBLK, L), lambda i: (0, i, 0))
_GR_SPEC = pl.BlockSpec((_EBLK, L), lambda i: (i + E_PAD // _EBLK, 0))


def _tc_node_encode(u8, Wn1p, bn1, Wn2, bn2, Wms, Wmr, bm):
    def body(u_r, w1_r, b1_r, w2_r, b2_r, wms_r, wmr_r, bm_r, h_r, p2_r):
        t = jnp.tanh(_dot(u_r[...], w1_r[...]) + b1_r[...])
        h = _dot(t, w2_r[...]) + b2_r[...]
        h_r[...] = h
        p2_r[0] = _dot(h, wms_r[...]) + bm_r[...]
        p2_r[1] = _dot(h, wmr_r[...])

    return pl.pallas_call(
        body,
        grid=(N_PAD // _NBLK,),
        in_specs=[_node_specs(8), _w_spec((8, L)), _w_spec((1, L)),
                  _w_spec((L, L)), _w_spec((1, L)), _w_spec((L, L)),
                  _w_spec((L, L)), _w_spec((1, L))],
        out_specs=[_node_specs(L), _P2_SPEC],
        out_shape=[jax.ShapeDtypeStruct((N_PAD, L), _F32),
                   jax.ShapeDtypeStruct((2, N_PAD, L), _F32)],
    )(u8, Wn1p, bn1, Wn2, bn2, Wms, Wmr, bm)


def _tc_edge_encode(x2g, We1p, be1, We2, be2):
    def body(xs_r, xr_r, w1_r, b1_r, w2_r, b2_r, e_r):
        z = xs_r[...] - xr_r[...]
        z = jnp.where(z >= 1.0, z - 2.0, z)
        z = jnp.where(z < -1.0, z + 2.0, z)
        d = jnp.sqrt(jnp.sum(z * z, axis=1, keepdims=True))
        lane = lax.broadcasted_iota(jnp.int32, z.shape, 1)
        za = jnp.where(lane == 2, d, z)
        f = _dot(za, w1_r[...]) + b1_r[...]
        e_r[...] = _dot(jnp.tanh(f), w2_r[...]) + b2_r[...]

    return pl.pallas_call(
        body,
        grid=(E_PAD // _EBLK,),
        in_specs=[_edge_specs(L), _GR_SPEC, _w_spec((L, L)),
                  _w_spec((1, L)), _w_spec((L, L)), _w_spec((1, L))],
        out_specs=_edge_specs(L),
        out_shape=jax.ShapeDtypeStruct((E_PAD, L), _F32),
    )(x2g, x2g, We1p, be1, We2, be2)


def _tc_edge_update(e, g2, Wme):
    def body(e_r, gs_r, gr_r, w_r, o_r):
        o_r[...] = e_r[...] + jnp.tanh(
            _dot(e_r[...], w_r[...]) + gs_r[...] + gr_r[...])

    return pl.pallas_call(
        body,
        grid=(E_PAD // _EBLK,),
        in_specs=[_edge_specs(L), _edge_specs(L), _GR_SPEC, _w_spec((L, L))],
        out_specs=_edge_specs(L),
        out_shape=jax.ShapeDtypeStruct((E_PAD, L), _F32),
    )(e, g2, g2, Wme)


def _tc_node_update(h, agg, Wuh, Wua, bu, Wms, Wmr, bm):
    def body(h_r, a0_r, a1_r, wh_r, wa_r, bu_r, wms_r, wmr_r, bm_r,
             hn_r, p2_r):
        hn = h_r[...] + jnp.tanh(
            _dot(h_r[...], wh_r[...]) + _dot(a0_r[0] + a1_r[0], wa_r[...])
            + bu_r[...])
        hn_r[...] = hn
        p2_r[0] = _dot(hn, wms_r[...]) + bm_r[...]
        p2_r[1] = _dot(hn, wmr_r[...])

    aspec = pl.BlockSpec((1, _NBLK, L), lambda i: (0, i, 0))
    return pl.pallas_call(
        body,
        grid=(N_PAD // _NBLK,),
        in_specs=[_node_specs(L), aspec, aspec, _w_spec((L, L)),
                  _w_spec((L, L)), _w_spec((1, L)), _w_spec((L, L)),
                  _w_spec((L, L)), _w_spec((1, L))],
        out_specs=[_node_specs(L), _P2_SPEC],
        out_shape=[jax.ShapeDtypeStruct((N_PAD, L), _F32),
                   jax.ShapeDtypeStruct((2, N_PAD, L), _F32)],
    )(h, agg[:1], agg[1:], Wuh, Wua, bu, Wms, Wmr, bm)


def _tc_node_final(h, agg, Wuh, Wua, bu, Wd1, bd1, Wd2p, bd2p):
    def body(h_r, a0_r, a1_r, wh_r, wa_r, bu_r, w1_r, b1_r, w2_r, b2_r, o_r):
        hn = h_r[...] + jnp.tanh(
            _dot(h_r[...], wh_r[...]) + _dot(a0_r[0] + a1_r[0], wa_r[...])
            + bu_r[...])
        t = jnp.tanh(_dot(hn, w1_r[...]) + b1_r[...])
        o_r[...] = _dot(t, w2_r[...]) + b2_r[...]

    aspec = pl.BlockSpec((1, _NBLK, L), lambda i: (0, i, 0))
    return pl.pallas_call(
        body,
        grid=(N_PAD // _NBLK,),
        in_specs=[_node_specs(L), aspec, aspec, _w_spec((L, L)),
                  _w_spec((L, L)), _w_spec((1, L)), _w_spec((L, L)),
                  _w_spec((1, L)), _w_spec((L, 8)), _w_spec((1, 8))],
        out_specs=_node_specs(8),
        out_shape=jax.ShapeDtypeStruct((N_PAD, 8), _F32),
    )(h, agg[:1], agg[1:], Wuh, Wua, bu, Wd1, bd1, Wd2p, bd2p)


def kernel(u_inp, x_inp, edge_index, params):
    p = params
    row = lambda v: v.reshape(1, -1)

    u8 = jnp.pad(u_inp[0, 0], ((0, N_PAD - N), (0, 8 - D_IN)))
    x128 = jnp.pad(x_inp, ((0, N_PAD - N), (0, L - D_X)))
    x2 = jnp.concatenate([x128, x128], axis=0)
    pad_e = E_PAD - E
    snd3 = jnp.pad(edge_index[0], (0, pad_e)).reshape(NW, C, K)
    rcv3 = jnp.pad(edge_index[1], (0, pad_e),
                   constant_values=N).reshape(NW, C, K)
    idx2 = jnp.concatenate([snd3, rcv3 + N_PAD], axis=1)

    Wn1p = jnp.pad(p["Wn1"], ((0, 8 - D_IN), (0, 0)))
    We1p = jnp.pad(p["We1"], ((0, L - D_X - 1), (0, 0)))
    Wm_e = p["Wm"][:, :L]
    Wm_s = p["Wm"][:, L:2 * L]
    Wm_r = p["Wm"][:, 2 * L:]
    Wu_h = p["Wu"][:, :L]
    Wu_a = p["Wu"][:, L:]
    Wd2p = jnp.pad(p["Wd2"], ((0, 0), (0, 8 - D_IN)))
    bd2p = row(jnp.pad(p["bd2"], (0, 8 - D_IN)))

    h, p2 = _tc_node_encode(u8, Wn1p, row(p["bn1"]), p["Wn2"],
                            row(p["bn2"]), Wm_s[0], Wm_r[0],
                            row(p["bm"][0]))
    x2g = _sc_gather2(x2, idx2)
    e = _tc_edge_encode(x2g, We1p, row(p["be1"]), p["We2"], row(p["be2"]))

    for s in range(STEPS):
        g2 = _sc_gather2(p2.reshape(2 * N_PAD, L), idx2)
        e = _tc_edge_update(e, g2, Wm_e[s])
        agg = _sc_segment_sum(e, rcv3)
        if s + 1 < STEPS:
            h, p2 = _tc_node_update(h, agg, Wu_h[s], Wu_a[s],
                                    row(p["bu"][s]), Wm_s[s + 1],
                                    Wm_r[s + 1], row(p["bm"][s + 1]))
        else:
            out = _tc_node_final(h, agg, Wu_h[s], Wu_a[s], row(p["bu"][s]),
                                 p["Wd1"], row(p["bd1"]), Wd2p, bd2p)

    return out[:N, :D_IN]


# speedup vs baseline: 1.3181x; 1.0227x over previous
import functools

import jax
import jax.numpy as jnp
from jax import lax
from jax.experimental import pallas as pl
from jax.experimental.pallas import tpu as pltpu
from jax.experimental.pallas import tpu_sc as plsc

N = 10000
E = 320000
D_IN = 3
D_X = 2
L = 128
STEPS = 6

N_PAD = 10240
E_PAD = 327680
NC = 2
NS = 16
NW = NC * NS
EPW = E_PAD // NW
K = 128
C = EPW // K
TOT = 2 * C
S = 5
RPS = N_PAD // NS
ZCH = RPS // K

_F32 = jnp.float32


def _mesh():
    return plsc.VectorSubcoreMesh(core_axis_name="c", subcore_axis_name="s",
                                  num_cores=NC, num_subcores=NS)


def _sc_gather2(t2, idx2):
    D = t2.shape[-1]

    @functools.partial(
        pl.kernel,
        out_type=jax.ShapeDtypeStruct((2 * E_PAD, D), _F32),
        mesh=_mesh(),
        scratch_types=[pltpu.VMEM((TOT, K), jnp.int32)]
        + [pltpu.VMEM((K, D), _F32)] * 4
        + [pltpu.SemaphoreType.DMA] * 4,
    )
    def k(t2_h, i2_h, out_h, iva, *rest):
        bufs = rest[:4]
        sems = rest[4:]
        wid = lax.axis_index("c") * NS + lax.axis_index("s")
        pltpu.sync_copy(i2_h.at[wid], iva)
        base_s = wid * EPW
        base_r = E_PAD + wid * EPW
        Q = C // 2

        def body(j, carry):
            cps = [pltpu.async_copy(t2_h.at[iva.at[j + c * Q]], bufs[c],
                                    sems[c]) for c in range(4)]
            offs = [base_s + j * K, base_s + (j + Q) * K,
                    base_r + j * K, base_r + (j + Q) * K]
            for c in range(4):
                cps[c].wait()
                pltpu.sync_copy(bufs[c], out_h.at[pl.ds(offs[c], K)])
            return carry

        lax.fori_loop(0, Q, body, 0)

    return k(t2, idx2)


def _sc_segment_sum(e_all, rcv3):
    SS = 2

    @functools.partial(
        pl.kernel,
        out_type=jax.ShapeDtypeStruct((NC, N_PAD, L), _F32),
        mesh=_mesh(),
        scratch_types=[pltpu.VMEM((C, K), jnp.int32)]
        + [pltpu.VMEM((K, L), _F32)] * SS
        + [pltpu.SemaphoreType.DMA] * (2 * SS)
        + [pltpu.VMEM_SHARED((N_PAD, L), _F32)],
    )
    def k(e_h, r_h, out_h, idxv, *rest):
        slots = rest[:SS]
        rsem = rest[SS:2 * SS]
        ssem = rest[2 * SS:3 * SS]
        acc = rest[3 * SS]
        cid = lax.axis_index("c")
        sid = lax.axis_index("s")
        wid = cid * NS + sid
        pltpu.sync_copy(r_h.at[wid], idxv)

        def zrow(i, carry):
            def zcol(j, c2):
                slots[0][i, pl.ds(j * 16, 16)] = jnp.zeros((16,), _F32)
                return c2
            return lax.fori_loop(0, L // 16, zcol, carry)

        lax.fori_loop(0, K, zrow, 0)
        row0 = sid * RPS

        def zc(i, carry):
            pltpu.sync_copy(slots[0], acc.at[pl.ds(row0 + i * K, K)])
            return carry

        lax.fori_loop(0, ZCH, zc, 0)
        plsc.subcore_barrier()

        ebase = wid * EPW

        def body(g, carry):
            rds = [pltpu.async_copy(
                e_h.at[pl.ds(ebase + (g * SS + b) * K, K)], slots[b],
                rsem[b]) for b in range(SS)]
            sds = []
            for b in range(SS):
                rds[b].wait()
                sds.append(pltpu.async_copy(
                    slots[b], acc.at[idxv.at[g * SS + b]], ssem[b],
                    add=True))
            for sd in sds:
                sd.wait()
            return carry

        lax.fori_loop(0, C // SS, body, 0)
        plsc.subcore_barrier()

        def outc(i, carry):
            pltpu.sync_copy(acc.at[pl.ds(row0 + i * K, K)], slots[0])
            pltpu.sync_copy(slots[0], out_h.at[cid, pl.ds(row0 + i * K, K)])
            return carry

        lax.fori_loop(0, ZCH, outc, 0)

    return k(e_all, rcv3)


def _dot(a, b):
    return jnp.dot(a, b, preferred_element_type=_F32)


_NBLK = 2048
_EBLK = 2048


def _node_specs(d):
    return pl.BlockSpec((_NBLK, d), lambda i: (i, 0))


def _edge_specs(d):
    return pl.BlockSpec((_EBLK, d), lambda i: (i, 0))


def _w_spec(shape):
    return pl.BlockSpec(shape, lambda i: tuple(0 for _ in shape))


_P2_SPEC = pl.BlockSpec((2, _NBLK, L), lambda i: (0, i, 0))
_GR_SPEC = pl.BlockSpec((_EBLK, L), lambda i: (i + E_PAD // _EBLK, 0))


def _tc_node_encode(u8, Wn1p, bn1, Wn2, bn2, Wms, Wmr, bm):
    def body(u_r, w1_r, b1_r, w2_r, b2_r, wms_r, wmr_r, bm_r, h_r, p2_r):
        t = jnp.tanh(_dot(u_r[...], w1_r[...]) + b1_r[...])
        h = _dot(t, w2_r[...]) + b2_r[...]
        h_r[...] = h
        p2_r[0] = _dot(h, wms_r[...]) + bm_r[...]
        p2_r[1] = _dot(h, wmr_r[...])

    return pl.pallas_call(
        body,
        grid=(N_PAD // _NBLK,),
        in_specs=[_node_specs(8), _w_spec((8, L)), _w_spec((1, L)),
                  _w_spec((L, L)), _w_spec((1, L)), _w_spec((L, L)),
                  _w_spec((L, L)), _w_spec((1, L))],
        out_specs=[_node_specs(L), _P2_SPEC],
        out_shape=[jax.ShapeDtypeStruct((N_PAD, L), _F32),
                   jax.ShapeDtypeStruct((2, N_PAD, L), _F32)],
    )(u8, Wn1p, bn1, Wn2, bn2, Wms, Wmr, bm)


def _tc_edge_encode(x2g, We1p, be1, We2, be2):
    def body(xs_r, xr_r, w1_r, b1_r, w2_r, b2_r, e_r):
        z = xs_r[...] - xr_r[...]
        z = jnp.where(z >= 1.0, z - 2.0, z)
        z = jnp.where(z < -1.0, z + 2.0, z)
        d = jnp.sqrt(jnp.sum(z * z, axis=1, keepdims=True))
        lane = lax.broadcasted_iota(jnp.int32, z.shape, 1)
        za = jnp.where(lane == 2, d, z)
        f = _dot(za, w1_r[...]) + b1_r[...]
        e_r[...] = _dot(jnp.tanh(f), w2_r[...]) + b2_r[...]

    return pl.pallas_call(
        body,
        grid=(E_PAD // _EBLK,),
        in_specs=[_edge_specs(L), _GR_SPEC, _w_spec((L, L)),
                  _w_spec((1, L)), _w_spec((L, L)), _w_spec((1, L))],
        out_specs=_edge_specs(L),
        out_shape=jax.ShapeDtypeStruct((E_PAD, L), _F32),
    )(x2g, x2g, We1p, be1, We2, be2)


def _tc_edge_update(e, g2, Wme):
    def body(e_r, gs_r, gr_r, w_r, o_r):
        o_r[...] = e_r[...] + jnp.tanh(
            _dot(e_r[...], w_r[...]) + gs_r[...] + gr_r[...])

    return pl.pallas_call(
        body,
        grid=(E_PAD // _EBLK,),
        in_specs=[_edge_specs(L), _edge_specs(L), _GR_SPEC, _w_spec((L, L))],
        out_specs=_edge_specs(L),
        out_shape=jax.ShapeDtypeStruct((E_PAD, L), _F32),
    )(e, g2, g2, Wme)


def _tc_node_update(h, agg, Wuh, Wua, bu, Wms, Wmr, bm):
    def body(h_r, a0_r, a1_r, wh_r, wa_r, bu_r, wms_r, wmr_r, bm_r,
             hn_r, p2_r):
        hn = h_r[...] + jnp.tanh(
            _dot(h_r[...], wh_r[...]) + _dot(a0_r[0] + a1_r[0], wa_r[...])
            + bu_r[...])
        hn_r[...] = hn
        p2_r[0] = _dot(hn, wms_r[...]) + bm_r[...]
        p2_r[1] = _dot(hn, wmr_r[...])

    aspec = pl.BlockSpec((1, _NBLK, L), lambda i: (0, i, 0))
    return pl.pallas_call(
        body,
        grid=(N_PAD // _NBLK,),
        in_specs=[_node_specs(L), aspec, aspec, _w_spec((L, L)),
                  _w_spec((L, L)), _w_spec((1, L)), _w_spec((L, L)),
                  _w_spec((L, L)), _w_spec((1, L))],
        out_specs=[_node_specs(L), _P2_SPEC],
        out_shape=[jax.ShapeDtypeStruct((N_PAD, L), _F32),
                   jax.ShapeDtypeStruct((2, N_PAD, L), _F32)],
    )(h, agg[:1], agg[1:], Wuh, Wua, bu, Wms, Wmr, bm)


def _tc_node_final(h, agg, Wuh, Wua, bu, Wd1, bd1, Wd2p, bd2p):
    def body(h_r, a0_r, a1_r, wh_r, wa_r, bu_r, w1_r, b1_r, w2_r, b2_r, o_r):
        hn = h_r[...] + jnp.tanh(
            _dot(h_r[...], wh_r[...]) + _dot(a0_r[0] + a1_r[0], wa_r[...])
            + bu_r[...])
        t = jnp.tanh(_dot(hn, w1_r[...]) + b1_r[...])
        o_r[...] = _dot(t, w2_r[...]) + b2_r[...]

    aspec = pl.BlockSpec((1, _NBLK, L), lambda i: (0, i, 0))
    return pl.pallas_call(
        body,
        grid=(N_PAD // _NBLK,),
        in_specs=[_node_specs(L), aspec, aspec, _w_spec((L, L)),
                  _w_spec((L, L)), _w_spec((1, L)), _w_spec((L, L)),
                  _w_spec((1, L)), _w_spec((L, 8)), _w_spec((1, 8))],
        out_specs=_node_specs(8),
        out_shape=jax.ShapeDtypeStruct((N_PAD, 8), _F32),
    )(h, agg[:1], agg[1:], Wuh, Wua, bu, Wd1, bd1, Wd2p, bd2p)


def kernel(u_inp, x_inp, edge_index, params):
    p = params
    row = lambda v: v.reshape(1, -1)

    u8 = jnp.pad(u_inp[0, 0], ((0, N_PAD - N), (0, 8 - D_IN)))
    x128 = jnp.pad(x_inp, ((0, N_PAD - N), (0, L - D_X)))
    x2 = jnp.concatenate([x128, x128], axis=0)
    pad_e = E_PAD - E
    snd3 = jnp.pad(edge_index[0], (0, pad_e)).reshape(NW, C, K)
    rcv3 = jnp.pad(edge_index[1], (0, pad_e),
                   constant_values=N).reshape(NW, C, K)
    idx2 = jnp.concatenate([snd3, rcv3 + N_PAD], axis=1)

    Wn1p = jnp.pad(p["Wn1"], ((0, 8 - D_IN), (0, 0)))
    We1p = jnp.pad(p["We1"], ((0, L - D_X - 1), (0, 0)))
    Wm_e = p["Wm"][:, :L]
    Wm_s = p["Wm"][:, L:2 * L]
    Wm_r = p["Wm"][:, 2 * L:]
    Wu_h = p["Wu"][:, :L]
    Wu_a = p["Wu"][:, L:]
    Wd2p = jnp.pad(p["Wd2"], ((0, 0), (0, 8 - D_IN)))
    bd2p = row(jnp.pad(p["bd2"], (0, 8 - D_IN)))

    h, p2 = _tc_node_encode(u8, Wn1p, row(p["bn1"]), p["Wn2"],
                            row(p["bn2"]), Wm_s[0], Wm_r[0],
                            row(p["bm"][0]))
    x2g = _sc_gather2(x2, idx2)
    e = _tc_edge_encode(x2g, We1p, row(p["be1"]), p["We2"], row(p["be2"]))

    for s in range(STEPS):
        g2 = _sc_gather2(p2.reshape(2 * N_PAD, L), idx2)
        e = _tc_edge_update(e, g2, Wm_e[s])
        agg = _sc_segment_sum(e, rcv3)
        if s + 1 < STEPS:
            h, p2 = _tc_node_update(h, agg, Wu_h[s], Wu_a[s],
                                    row(p["bu"][s]), Wm_s[s + 1],
                                    Wm_r[s + 1], row(p["bm"][s + 1]))
        else:
            out = _tc_node_final(h, agg, Wu_h[s], Wu_a[s], row(p["bu"][s]),
                                 p["Wd1"], row(p["bd1"]), Wd2p, bd2p)

    return out[:N, :D_IN]
